# baseline (speedup 1.0000x reference)
import jax
import jax.numpy as jnp
from jax.experimental import pallas as pl
from jax.experimental.pallas import tpu as pltpu

_KTOP = 2
_NEG = float(jnp.finfo(jnp.float32).min)
_BB = 1024


def _gating(logits, E):
    eidx = jax.lax.broadcasted_iota(jnp.int32, logits.shape, 1)
    m1 = jnp.max(logits, axis=-1, keepdims=True)
    i1 = jnp.min(jnp.where(logits == m1, eidx, E), axis=-1, keepdims=True)
    l2 = jnp.where(eidx == i1, _NEG, logits)
    m2 = jnp.max(l2, axis=-1, keepdims=True)
    i2 = jnp.min(jnp.where(l2 == m2, eidx, E), axis=-1, keepdims=True)
    mask = (eidx == i1) | (eidx == i2)
    p = jnp.exp(logits - m1)
    pm = jnp.where(mask, p, 0.0)
    return pm / jnp.sum(pm, axis=-1, keepdims=True)


def _moe_pipe(x_hbm, wg_ref, bg_ref, w1_ref, b1_ref, w2_ref, b2_ref,
              sel_ref, out_hbm,
              xbuf, obuf, w1s, w2s, b1s, sem_in, sem_out):
    E, D, H = w1_ref.shape
    B = x_hbm.shape[0]
    n = B // _BB

    for e in range(E):
        w1s[:, e * H:(e + 1) * H] = w1_ref[e].astype(jnp.bfloat16)
        w2s[e * H:(e + 1) * H, :] = w2_ref[e].astype(jnp.bfloat16)
        b1s[:, e * H:(e + 1) * H] = b1_ref[e:e + 1, :]

    def in_copy(i, s):
        return pltpu.make_async_copy(
            x_hbm.at[pl.ds(i * _BB, _BB), :], xbuf.at[s], sem_in.at[s])

    def out_copy(i, s):
        return pltpu.make_async_copy(
            obuf.at[s], out_hbm.at[pl.ds(i * _BB, _BB), :], sem_out.at[s])

    in_copy(0, 0).start()
    for i in range(n):
        s = i % 2
        if i + 1 < n:
            in_copy(i + 1, 1 - s).start()
        in_copy(i, s).wait()
        if i >= 2:
            out_copy(i - 2, s).wait()

        x = xbuf[s]
        logits = jnp.dot(x, wg_ref[...], preferred_element_type=jnp.float32)
        logits = logits + bg_ref[...]
        g = _gating(logits, E)
        xb = x.astype(jnp.bfloat16)
        h = jnp.dot(xb, w1s[...], preferred_element_type=jnp.float32)
        h = h + b1s[...]
        hs = h * (1.0 + jax.lax.erf(h * 0.7071067811865476))
        g_exp = jnp.dot(g, sel_ref[...], preferred_element_type=jnp.float32)
        hg = (hs * g_exp).astype(jnp.bfloat16)
        out = jnp.dot(hg, w2s[...], preferred_element_type=jnp.float32)
        out = out + jnp.dot(g, b2_ref[...], preferred_element_type=jnp.float32)
        obuf[s] = out

        out_copy(i, s).start()
    out_copy(n - 2, (n - 2) % 2).wait()
    out_copy(n - 1, (n - 1) % 2).wait()


def kernel(x, Wg, bg, W1, b1, W2, b2):
    B, D = x.shape
    E = Wg.shape[-1]
    H = W1.shape[-1]
    bg2 = bg.reshape(1, E)
    sel = 0.5 * (jnp.arange(E * H)[None, :] // H
                 == jnp.arange(E)[:, None]).astype(jnp.float32)

    vmem = lambda shape: pl.BlockSpec(shape, lambda: (0,) * len(shape))
    out = pl.pallas_call(
        _moe_pipe,
        in_specs=[
            pl.BlockSpec(memory_space=pl.ANY),
            vmem((D, E)),
            vmem((1, E)),
            vmem((E, D, H)),
            vmem((E, H)),
            vmem((E, H, D)),
            vmem((E, D)),
            vmem((E, E * H)),
        ],
        out_specs=pl.BlockSpec(memory_space=pl.ANY),
        out_shape=jax.ShapeDtypeStruct((B, D), jnp.float32),
        scratch_shapes=[
            pltpu.VMEM((2, _BB, D), jnp.float32),
            pltpu.VMEM((2, _BB, D), jnp.float32),
            pltpu.VMEM((D, E * H), jnp.bfloat16),
            pltpu.VMEM((E * H, D), jnp.bfloat16),
            pltpu.VMEM((1, E * H), jnp.float32),
            pltpu.SemaphoreType.DMA((2,)),
            pltpu.SemaphoreType.DMA((2,)),
        ],
    )(x, Wg, bg2, W1, b1, W2, b2, sel)
    return out
